# Initial kernel scaffold; baseline (speedup 1.0000x reference)
import jax
import jax.numpy as jnp
from jax.experimental import pallas as pl


def kernel(point_seq, feat_seq):
    raise NotImplementedError("write your pallas kernel here")


# speedup vs baseline: 20.0562x; 20.0562x over previous
import functools

import jax
import jax.numpy as jnp
from jax import lax
from jax.experimental import pallas as pl
from jax.experimental.pallas import tpu as pltpu
from jax.experimental.pallas import tpu_sc as plsc

K = 16
N = 2048
DF = 64
PW = 16

ROWS = 256


def _knn_tc_kernel(x2_ref, x1t_ref, dist_ref, idx_ref):
    x2 = x2_ref[0]
    x1t = x1t_ref[0]
    n2 = jnp.sum(x2 * x2, axis=1, keepdims=True)
    n1 = jnp.sum(x1t * x1t, axis=0, keepdims=True)
    cross = jnp.dot(x2, x1t, preferred_element_type=jnp.float32)
    d2 = n2 + n1 - 2.0 * cross
    col = lax.broadcasted_iota(jnp.int32, (ROWS, N), 1)
    dcols = []
    icols = []
    for _ in range(K):
        m = jnp.min(d2, axis=1, keepdims=True)
        sel = jnp.min(jnp.where(d2 == m, col, N), axis=1, keepdims=True)
        dcols.append(m)
        icols.append(sel)
        d2 = jnp.where(col == sel, jnp.float32(jnp.inf), d2)
    dist_ref[0] = jnp.concatenate(dcols, axis=1)
    idx_ref[0] = jnp.concatenate(icols, axis=1)


def _knn_all_frames(x2, x1t):
    f = x2.shape[0]
    grid = (f, N // ROWS)
    return pl.pallas_call(
        _knn_tc_kernel,
        grid=grid,
        in_specs=[
            pl.BlockSpec((1, ROWS, 3), lambda i, r: (i, r, 0)),
            pl.BlockSpec((1, 3, N), lambda i, r: (i, 0, 0)),
        ],
        out_specs=[
            pl.BlockSpec((1, ROWS, K), lambda i, r: (i, r, 0)),
            pl.BlockSpec((1, ROWS, K), lambda i, r: (i, r, 0)),
        ],
        out_shape=[
            jax.ShapeDtypeStruct((f, N, K), jnp.float32),
            jax.ShapeDtypeStruct((f, N, K), jnp.int32),
        ],
    )(x2, x1t)


def _sc_chain_gather(idx_tbl, idx_col0, pts_tbl, feats_tbl, frames):
    info = plsc.get_sparse_core_info()
    nc, ns = info.num_cores, info.num_subcores
    nw = nc * ns
    spw = N // nw
    mesh = plsc.VectorSubcoreMesh(core_axis_name="c", subcore_axis_name="s")

    @functools.partial(
        pl.kernel,
        mesh=mesh,
        compiler_params=pltpu.CompilerParams(use_tc_tiling_on_sc=False),
        out_type=(
            jax.ShapeDtypeStruct((frames * N, K), jnp.int32),
            jax.ShapeDtypeStruct((frames * N * K, PW), jnp.float32),
            jax.ShapeDtypeStruct((frames * N * K, DF), jnp.float32),
        ),
        scratch_types=[
            pltpu.VMEM((spw, K), jnp.int32),
            pltpu.VMEM((spw,), jnp.int32),
            pltpu.VMEM((spw,), jnp.int32),
            pltpu.VMEM((spw * K,), jnp.int32),
            pltpu.VMEM((spw * K, PW), jnp.float32),
            pltpu.VMEM((spw * K, DF), jnp.float32),
            pltpu.SemaphoreType.DMA,
            pltpu.SemaphoreType.DMA,
        ],
    )
    def chain_kernel(idx_hbm, col0_hbm, pts_hbm, feats_hbm,
                     patch_hbm, ppts_hbm, pfeats_hbm,
                     rows_v, c_v, craw_v, gidx_v, pbuf_v, fbuf_v, sem, sem2):
        wid = lax.axis_index("s") * nc + lax.axis_index("c")
        wbase = wid * spw
        for f in range(frames):
            if f == 0:
                pltpu.sync_copy(idx_hbm.at[pl.ds(wbase, spw)], rows_v)
                pltpu.sync_copy(col0_hbm.at[pl.ds(wbase, spw)], craw_v)
            else:
                pltpu.async_copy(idx_hbm.at[c_v], rows_v, sem).wait()
                pltpu.async_copy(col0_hbm.at[c_v], craw_v, sem2).wait()
            pltpu.sync_copy(rows_v, patch_hbm.at[pl.ds(f * N + wbase, spw)])

            def build_row(g, _, f=f):
                gidx_v[pl.ds(g * K, K)] = rows_v[g] + jnp.int32(f * N)
                return 0
            lax.fori_loop(0, spw, build_row, 0)
            for b2 in range(spw // 16):
                c_v[pl.ds(16 * b2, 16)] = (
                    craw_v[pl.ds(16 * b2, 16)] + jnp.int32((f + 1) * N))

            pltpu.async_copy(feats_hbm.at[gidx_v], fbuf_v, sem).wait()
            pltpu.sync_copy(
                fbuf_v, pfeats_hbm.at[pl.ds((f * N + wbase) * K, spw * K)])
            pltpu.async_copy(pts_hbm.at[gidx_v], pbuf_v, sem).wait()
            pltpu.sync_copy(
                pbuf_v, ppts_hbm.at[pl.ds((f * N + wbase) * K, spw * K)])

    return chain_kernel(idx_tbl, idx_col0, pts_tbl, feats_tbl)


def kernel(point_seq, feat_seq):
    b, t, n, d = point_seq.shape
    d_feat = feat_seq.shape[-1]
    frames = b * t
    x1 = point_seq.reshape(frames, n, d)
    x2 = jnp.concatenate([point_seq[:, :1], point_seq], axis=1)[:, :-1]
    x2 = x2.reshape(frames, n, d)
    x1t = x1.transpose(0, 2, 1)

    dist, idx = _knn_all_frames(x2, x1t)

    idx_tbl = idx.reshape(frames * n, K)
    idx_col0 = idx_tbl[:, 0]
    pts_tbl = jnp.pad(x1.reshape(frames * n, d), ((0, 0), (0, PW - d)))
    feats_tbl = feat_seq.reshape(frames * n, d_feat)

    patchlets, ppoints, pfeats = _sc_chain_gather(
        idx_tbl, idx_col0, pts_tbl, feats_tbl, frames)

    return {
        "idx": idx.reshape(b, t, n, K),
        "distances": dist.reshape(b, t, n, K),
        "patchlets": patchlets.reshape(b, t, n, K),
        "patchlet_points": ppoints.reshape(b, t, n, K, PW)[..., :d],
        "patchlet_feats": pfeats.reshape(b, t, n, K, d_feat),
    }
